# baseline (speedup 1.0000x reference)
import functools
import jax
import jax.numpy as jnp
import numpy as np
from jax import lax
from jax.experimental import pallas as pl
from jax.experimental.pallas import tpu as pltpu
from jax.experimental.pallas import tpu_sc as plsc

F = 26
D = 16
HID = 32
B = 4096
K = 32
ROWS_PER_FIELD = 104000
NPAIR_COLS = (F - 1) * F
NW = 32
GROWS = F * B
BPW = GROWS // NW
NBLK = 8
BLK = B // NBLK


def _sc_body(tab_ref, eidx_ref, linw_ref, lidx_ref, emb_out, lin_out,
             idx_v, rows_v, lidx_v, lrows_v, sem_a, sem_b):
    wid = lax.axis_index("s") * 2 + lax.axis_index("c")
    base = wid * BPW
    pltpu.sync_copy(eidx_ref.at[pl.ds(base, BPW)], idx_v)
    pltpu.sync_copy(lidx_ref.at[pl.ds(base, BPW)], lidx_v)
    g_emb = pltpu.async_copy(tab_ref.at[idx_v], rows_v, sem_a)
    g_lin = pltpu.async_copy(linw_ref.at[lidx_v], lrows_v, sem_b)
    g_emb.wait()
    pltpu.sync_copy(rows_v, emb_out.at[pl.ds(base, BPW)])
    g_lin.wait()
    pltpu.sync_copy(lrows_v, lin_out.at[pl.ds(base, BPW)])


@functools.lru_cache(maxsize=1)
def _sc_gather():
    return pl.kernel(
        _sc_body,
        out_type=(
            jax.ShapeDtypeStruct((GROWS, D), jnp.float32),
            jax.ShapeDtypeStruct((GROWS, 1), jnp.float32),
        ),
        mesh=plsc.VectorSubcoreMesh(core_axis_name="c", subcore_axis_name="s"),
        scratch_types=[
            pltpu.VMEM((BPW,), jnp.int32),
            pltpu.VMEM((BPW, D), jnp.float32),
            pltpu.VMEM((BPW,), jnp.int32),
            pltpu.VMEM((BPW, 1), jnp.float32),
            pltpu.SemaphoreType.DMA,
            pltpu.SemaphoreType.DMA,
        ],
        compiler_params=pltpu.CompilerParams(use_tc_tiling_on_sc=False),
    )


def _tc_scores_body(emb_ref, w1_ref, b1_ref, w2_ref, scores_ref, colsum_ref):
    step = pl.program_id(0)
    E3 = emb_ref[...]
    Ef = E3.reshape(F * BLK, D)
    w1a = w1_ref[0:D, :]
    w1b = w1_ref[D:2 * D, :]
    U3 = jnp.dot(Ef, w1a, preferred_element_type=jnp.float32).reshape(F, BLK, HID)
    V3 = jnp.dot(Ef, w1b, preferred_element_type=jnp.float32).reshape(F, BLK, HID)
    U = jnp.concatenate([U3[f] for f in range(F)], axis=1)
    V = jnp.concatenate([V3[f] for f in range(F)], axis=1)
    b1t = jnp.concatenate([b1_ref[...]] * F, axis=1)
    w2t = jnp.concatenate([w2_ref[...]] * F, axis=0)
    r_io = lax.broadcasted_iota(jnp.int32, (F * HID, F), 0)
    c_io = lax.broadcasted_iota(jnp.int32, (F * HID, F), 1)
    G2 = jnp.where(r_io // HID == c_io, w2t, 0.0)
    parts = []
    for d in range(1, F):
        Vs = jnp.concatenate(
            [V[:, d * HID:], jnp.zeros((BLK, d * HID), jnp.float32)], axis=1)
        Hd = jax.nn.relu(U + Vs + b1t)
        Sd = jnp.dot(Hd, G2, preferred_element_type=jnp.float32)
        scores_ref[:, (d - 1) * F:d * F] = Sd
        parts.append(jnp.sum(Sd, axis=0, keepdims=True))
    cs = jnp.concatenate(parts, axis=1)

    @pl.when(step == 0)
    def _():
        colsum_ref[...] = cs

    @pl.when(step != 0)
    def _():
        colsum_ref[...] = colsum_ref[...] + cs


def _tc_scores(emb_fm, w1, b1r, w2):
    return pl.pallas_call(
        _tc_scores_body,
        grid=(NBLK,),
        in_specs=[
            pl.BlockSpec((F, BLK, D), lambda i: (0, i, 0)),
            pl.BlockSpec((2 * D, HID), lambda i: (0, 0)),
            pl.BlockSpec((1, HID), lambda i: (0, 0)),
            pl.BlockSpec((HID, 1), lambda i: (0, 0)),
        ],
        out_specs=[
            pl.BlockSpec((BLK, NPAIR_COLS), lambda i: (i, 0)),
            pl.BlockSpec((1, NPAIR_COLS), lambda i: (0, 0)),
        ],
        out_shape=[
            jax.ShapeDtypeStruct((B, NPAIR_COLS), jnp.float32),
            jax.ShapeDtypeStruct((1, NPAIR_COLS), jnp.float32),
        ],
    )(emb_fm, w1, b1r, w2)


def _tc_mask_body(cs_ref, mask_ref):
    cs = cs_ref[...]
    c = lax.broadcasted_iota(jnp.int32, (1, NPAIR_COLS), 1)
    dd = c // F + 1
    ii = c % F
    valid = ii < (F - dd)
    neg = jnp.float32(-3e38)
    vals = jnp.where(valid, cs, neg)
    m = jnp.zeros((1, NPAIR_COLS), jnp.float32)
    for _ in range(K):
        cur = jnp.max(vals)
        idxs = jnp.where(vals == cur, c, NPAIR_COLS)
        mi = jnp.min(idxs)
        newly = c == mi
        m = jnp.where(newly, 1.0, m)
        vals = jnp.where(newly, neg, vals)
    mask_ref[...] = m


def _tc_mask(colsum):
    return pl.pallas_call(
        _tc_mask_body,
        out_shape=jax.ShapeDtypeStruct((1, NPAIR_COLS), jnp.float32),
    )(colsum)


def _tc_final_body(emb_ref, s_ref, mask_ref, linv_ref, b2_ref, linb_ref,
                   pw_ref, pb_ref, out_ref):
    E3 = emb_ref[...]
    EA = jnp.concatenate([E3[f] for f in range(F)], axis=1)
    w = (s_ref[...] + b2_ref[0, 0]) * mask_ref[...]
    r_io = lax.broadcasted_iota(jnp.int32, (F, F * D), 0)
    c_io = lax.broadcasted_iota(jnp.int32, (F, F * D), 1)
    expand = jnp.where(c_io // D == r_io, 1.0, 0.0)
    acc = jnp.zeros((BLK, F * D), jnp.float32)
    for d in range(1, F):
        ES = jnp.concatenate(
            [EA[:, d * D:], jnp.zeros((BLK, d * D), jnp.float32)], axis=1)
        wd = w[:, (d - 1) * F:d * F]
        wexp = jnp.dot(wd, expand, preferred_element_type=jnp.float32)
        acc = acc + EA * ES * wexp
    summed = jnp.zeros((BLK, D), jnp.float32)
    for f in range(F):
        summed = summed + acc[:, f * D:(f + 1) * D]
    cross = jnp.dot(summed, pw_ref[...],
                    preferred_element_type=jnp.float32) + pb_ref[0, 0]
    lin = jnp.sum(linv_ref[...], axis=1, keepdims=True) + linb_ref[0, 0]
    out_ref[...] = jax.nn.sigmoid(lin + cross)


def _tc_final(emb_fm, scores, mask, lin_bf, b2r, linbr, proj_w, pbr):
    return pl.pallas_call(
        _tc_final_body,
        grid=(NBLK,),
        in_specs=[
            pl.BlockSpec((F, BLK, D), lambda i: (0, i, 0)),
            pl.BlockSpec((BLK, NPAIR_COLS), lambda i: (i, 0)),
            pl.BlockSpec((1, NPAIR_COLS), lambda i: (0, 0)),
            pl.BlockSpec((BLK, F), lambda i: (i, 0)),
            pl.BlockSpec((1, 1), lambda i: (0, 0)),
            pl.BlockSpec((1, 1), lambda i: (0, 0)),
            pl.BlockSpec((D, 1), lambda i: (0, 0)),
            pl.BlockSpec((1, 1), lambda i: (0, 0)),
        ],
        out_specs=pl.BlockSpec((BLK, 1), lambda i: (i, 0)),
        out_shape=jax.ShapeDtypeStruct((B, 1), jnp.float32),
    )(emb_fm, scores, mask, lin_bf, b2r, linbr, proj_w, pbr)


@jax.jit
def kernel(x, tables, lin_w, lin_b, w1, b1, w2, b2, proj_w, proj_b):
    tab_flat = tables.reshape(F * ROWS_PER_FIELD, D)
    emb_off = (np.arange(F, dtype=np.int32) * (ROWS_PER_FIELD + 4000))[:, None]
    lin_off = (np.arange(F, dtype=np.int32) * 4000)[None, :]
    emb_idx = (x.T + emb_off).reshape(-1)
    lin_idx = (x + lin_off).reshape(-1)
    emb_flat, lin_vals = _sc_gather()(tab_flat, emb_idx, lin_w, lin_idx)
    emb_fm = emb_flat.reshape(F, B, D)
    lin_bf = lin_vals.reshape(B, F)
    scores, colsum = _tc_scores(emb_fm, w1, b1.reshape(1, HID), w2)
    mask = _tc_mask(colsum)
    out = _tc_final(emb_fm, scores, mask, lin_bf, b2.reshape(1, 1),
                    lin_b.reshape(1, 1), proj_w, proj_b.reshape(1, 1))
    return out
